# baseline (speedup 1.0000x reference)
import jax
import jax.numpy as jnp
from jax import lax
from jax.experimental import pallas as pl
from jax.experimental.pallas import tpu as pltpu

_NX = 4


def _body(id_ref, x_hbm, w_hbm, out_hbm, wbuf, slab, sem_x, sem_r, sem_s):
    B, C, S, H, _ = x_hbm.shape

    def x_copy(b, k):
        return pltpu.make_async_copy(
            x_hbm.at[b], out_hbm.at[b, pl.ds(0, C)], sem_x.at[k]
        )

    def row_fetch(b, p):
        grp = 8 * (id_ref[b] // 8)
        return pltpu.make_async_copy(
            w_hbm.at[pl.ds(grp, 8), :], wbuf.at[p], sem_r.at[p]
        )

    def slab_copy(b, p):
        return pltpu.make_async_copy(slab.at[p], out_hbm.at[b, C], sem_s.at[p])

    row_fetch(0, 0).start()

    def step(b, carry):
        p = lax.rem(b, 2)
        k = lax.rem(b, _NX)

        @pl.when(b >= _NX)
        def _():
            x_copy(b - _NX, k).wait()

        x_copy(b, k).start()

        @pl.when(b + 1 < B)
        def _():
            row_fetch(b + 1, 1 - p).start()

        row_fetch(b, p).wait()
        row = id_ref[b] % 8
        w64 = wbuf[p, pl.ds(row, 1), :].reshape(H, H)

        @pl.when(b >= 2)
        def _():
            slab_copy(b - 2, p).wait()

        for t in range(S):
            slab[p, t] = w64
        slab_copy(b, p).start()
        return carry

    lax.fori_loop(0, B, step, 0)

    slab_copy(B - 2, 0).wait()
    slab_copy(B - 1, 1).wait()
    for j in range(_NX):
        x_copy(B - _NX + j, lax.rem(B - _NX + j, _NX)).wait()


def kernel(x, id, W):
    b, c, s, h, _ = x.shape
    return pl.pallas_call(
        _body,
        in_specs=[
            pl.BlockSpec(memory_space=pltpu.SMEM),
            pl.BlockSpec(memory_space=pltpu.MemorySpace.HBM),
            pl.BlockSpec(memory_space=pltpu.MemorySpace.HBM),
        ],
        out_specs=pl.BlockSpec(memory_space=pltpu.MemorySpace.HBM),
        out_shape=jax.ShapeDtypeStruct((b, c + 1, s, h, h), x.dtype),
        scratch_shapes=[
            pltpu.VMEM((2, 8, h * h), jnp.float32),
            pltpu.VMEM((2, s, h, h), jnp.float32),
            pltpu.SemaphoreType.DMA((_NX,)),
            pltpu.SemaphoreType.DMA((2,)),
            pltpu.SemaphoreType.DMA((2,)),
        ],
    )(id, x, W)


# speedup vs baseline: 12.9080x; 12.9080x over previous
import jax
import jax.numpy as jnp
from jax import lax
from jax.experimental import pallas as pl
from jax.experimental.pallas import tpu as pltpu

_BB = 8


def _body(id_ref, x_ref, w_hbm, out_ref, wrows, sem):
    i = pl.program_id(0)
    n = pl.num_programs(0)
    c = x_ref.shape[1]
    s = x_ref.shape[2]
    h = x_ref.shape[3]

    def fetch(g, p):
        for j in range(_BB):
            pltpu.make_async_copy(
                w_hbm.at[pl.ds(id_ref[g * _BB + j], 1), :],
                wrows.at[p, pl.ds(j, 1)],
                sem.at[p],
            ).start()

    def drain(p):
        for j in range(_BB):
            pltpu.make_async_copy(
                w_hbm.at[pl.ds(0, 1), :], wrows.at[p, pl.ds(j, 1)], sem.at[p]
            ).wait()

    p = lax.rem(i, 2)

    @pl.when(i == 0)
    def _():
        fetch(0, 0)

    @pl.when(i + 1 < n)
    def _():
        fetch(i + 1, 1 - p)

    out_ref[:, :c] = x_ref[...]
    drain(p)
    for j in range(_BB):
        w64 = wrows[p, pl.ds(j, 1), :].reshape(h, h)
        for t in range(s):
            out_ref[j, c, t] = w64


def kernel(x, id, W):
    b, c, s, h, _ = x.shape
    grid_spec = pltpu.PrefetchScalarGridSpec(
        num_scalar_prefetch=1,
        grid=(b // _BB,),
        in_specs=[
            pl.BlockSpec((_BB, c, s, h, h), lambda i, idr: (i, 0, 0, 0, 0)),
            pl.BlockSpec(memory_space=pltpu.MemorySpace.HBM),
        ],
        out_specs=pl.BlockSpec((_BB, c + 1, s, h, h), lambda i, idr: (i, 0, 0, 0, 0)),
        scratch_shapes=[
            pltpu.VMEM((2, _BB, h * h), jnp.float32),
            pltpu.SemaphoreType.DMA((2,)),
        ],
    )
    return pl.pallas_call(
        _body,
        grid_spec=grid_spec,
        out_shape=jax.ShapeDtypeStruct((b, c + 1, s, h, h), x.dtype),
    )(id, x, W)


# speedup vs baseline: 12.9268x; 1.0015x over previous
import jax
import jax.numpy as jnp
from jax import lax
from jax.experimental import pallas as pl
from jax.experimental.pallas import tpu as pltpu

_BB = 8
_K = 3


def _body(id_ref, x_hbm, w_hbm, out_hbm, obuf, wrows, sem_in, sem_out, sem_row):
    B, C, S, H, _ = x_hbm.shape
    G = B // _BB

    def in_start(g, slot):
        for c in range(C):
            pltpu.make_async_copy(
                x_hbm.at[pl.ds(g * _BB, _BB), c],
                obuf.at[slot, :, c],
                sem_in.at[slot, c],
            ).start()

    def in_wait(slot):
        for c in range(C):
            pltpu.make_async_copy(
                x_hbm.at[pl.ds(0, _BB), c], obuf.at[slot, :, c], sem_in.at[slot, c]
            ).wait()

    def out_start(g, slot):
        for c in range(C + 1):
            pltpu.make_async_copy(
                obuf.at[slot, :, c],
                out_hbm.at[pl.ds(g * _BB, _BB), c],
                sem_out.at[slot, c],
            ).start()

    def out_wait(slot):
        for c in range(C + 1):
            pltpu.make_async_copy(
                obuf.at[slot, :, c],
                out_hbm.at[pl.ds(0, _BB), c],
                sem_out.at[slot, c],
            ).wait()

    def rows_start(g, p):
        for j in range(_BB):
            pltpu.make_async_copy(
                w_hbm.at[pl.ds(id_ref[g * _BB + j], 1), :],
                wrows.at[p, pl.ds(j, 1)],
                sem_row.at[p],
            ).start()

    def rows_wait(p):
        for j in range(_BB):
            pltpu.make_async_copy(
                w_hbm.at[pl.ds(0, 1), :], wrows.at[p, pl.ds(j, 1)], sem_row.at[p]
            ).wait()

    def step(g, carry):
        slot = lax.rem(g, _K)
        prev = lax.rem(g - 1 + _K, _K)
        p = lax.rem(g, 2)

        @pl.when(g < G)
        def _():
            @pl.when(g >= _K)
            def _():
                out_wait(slot)

            in_start(g, slot)
            rows_start(g, p)

        @pl.when(g >= 1)
        def _():
            in_wait(prev)
            rows_wait(1 - p)
            for j in range(_BB):
                w64 = wrows[1 - p, pl.ds(j, 1), :].reshape(H, H)
                for t in range(S):
                    obuf[prev, j, C, t] = w64
            out_start(g - 1, prev)

        return carry

    lax.fori_loop(0, G + 1, step, 0)

    for j in range(1, _K + 1):
        out_wait((G - j) % _K)


def kernel(x, id, W):
    b, c, s, h, _ = x.shape
    return pl.pallas_call(
        _body,
        in_specs=[
            pl.BlockSpec(memory_space=pltpu.SMEM),
            pl.BlockSpec(memory_space=pltpu.MemorySpace.HBM),
            pl.BlockSpec(memory_space=pltpu.MemorySpace.HBM),
        ],
        out_specs=pl.BlockSpec(memory_space=pltpu.MemorySpace.HBM),
        out_shape=jax.ShapeDtypeStruct((b, c + 1, s, h, h), x.dtype),
        scratch_shapes=[
            pltpu.VMEM((_K, _BB, c + 1, s, h, h), jnp.float32),
            pltpu.VMEM((2, _BB, h * h), jnp.float32),
            pltpu.SemaphoreType.DMA((_K, c)),
            pltpu.SemaphoreType.DMA((_K, c + 1)),
            pltpu.SemaphoreType.DMA((2,)),
        ],
    )(id, x, W)
